# baseline (speedup 1.0000x reference)
import functools

import jax
import jax.numpy as jnp
import numpy as np
from jax import lax
from jax.experimental import pallas as pl
from jax.experimental.pallas import tpu as pltpu
from jax.experimental.pallas import tpu_sc as plsc

_LANES = 16
_RING = 4
_CB = 8


@functools.lru_cache(maxsize=None)
def _build_sc_call(n, d):
    info = plsc.get_sparse_core_info()
    nc = info.num_cores
    nw = nc * info.num_subcores
    bpw = n // nw
    cb = _CB
    steps = bpw // cb
    assert steps % _RING == 0
    mesh = plsc.VectorSubcoreMesh(core_axis_name="c", subcore_axis_name="s")

    @functools.partial(
        pl.kernel,
        mesh=mesh,
        out_type=jax.ShapeDtypeStruct((n, 2 * d), jnp.float32),
        scratch_types=[
            pltpu.VMEM((bpw,), jnp.int32),
            pltpu.VMEM((_RING, cb, d), jnp.float32),
            pltpu.VMEM((_RING, cb, d), jnp.float32),
            pltpu.VMEM((_RING, cb, d), jnp.float32),
            pltpu.SemaphoreType.DMA((_RING,)),
            pltpu.SemaphoreType.DMA((_RING,)),
            pltpu.SemaphoreType.DMA((_RING,)),
            pltpu.SemaphoreType.DMA((_RING,)),
        ],
    )
    def call(x_hbm, g_hbm, out_hbm, idx_all, av, xv, dv, gsem, xsem, wasem, wdsem):
        wid = lax.axis_index("s") * nc + lax.axis_index("c")
        base = wid * bpw
        pltpu.sync_copy(g_hbm.at[pl.ds(base, bpw)], idx_all)

        def issue_loads(i, b):
            r0 = base + i * cb
            pltpu.async_copy(x_hbm.at[pl.ds(r0, cb)],
                             av.at[b], gsem.at[b])
            pltpu.async_copy(x_hbm.at[pl.ds(r0, cb)], xv.at[b], xsem.at[b])

        def wait_writes(b):
            pltpu.make_async_copy(
                av.at[b], out_hbm.at[pl.ds(0, cb), pl.ds(0, d)], wasem.at[b]
            ).wait()
            pltpu.make_async_copy(
                dv.at[b], out_hbm.at[pl.ds(0, cb), pl.ds(d, d)], wdsem.at[b]
            ).wait()

        for b in range(_RING - 1):
            issue_loads(b, b)

        def group(k, carry):
            for b in range(_RING):
                i = _RING * k + b
                bp = (b + _RING - 1) % _RING

                @pl.when(jnp.logical_and(i >= 1, i + (_RING - 1) < steps))
                def _():
                    wait_writes(bp)

                @pl.when(i + (_RING - 1) < steps)
                def _():
                    issue_loads(i + (_RING - 1), bp)

                pltpu.make_async_copy(
                    x_hbm.at[pl.ds(0, cb)], av.at[b], gsem.at[b]).wait()
                pltpu.make_async_copy(
                    x_hbm.at[pl.ds(0, cb)], xv.at[b], xsem.at[b]).wait()

                def row(r, rc):
                    for c in range(d // _LANES):
                        sl = pl.ds(c * _LANES, _LANES)
                        dv[b, r, sl] = xv[b, r, sl] - av[b, r, sl]
                    return rc

                lax.fori_loop(0, cb, row, 0)

                r0 = base + i * cb
                pltpu.async_copy(
                    av.at[b], out_hbm.at[pl.ds(r0, cb), pl.ds(0, d)], wasem.at[b])
                pltpu.async_copy(
                    dv.at[b], out_hbm.at[pl.ds(r0, cb), pl.ds(d, d)], wdsem.at[b])
            return carry

        lax.fori_loop(0, steps // _RING, group, 0)
        for b in range(_RING):
            wait_writes(b)

    return call


@functools.lru_cache(maxsize=None)
def _anchor_indices(n):
    with jax.ensure_compile_time_eval(), \
            jax.default_device(jax.devices("cpu")[0]):
        k1, k2 = jax.random.split(jax.random.key(42))
        perm = jax.random.permutation(k1, n)
        idx = jax.random.randint(k2, (n,), 0, n)
        g = jnp.take(perm, idx, axis=0)
        return np.asarray(g, dtype=np.int32)


def kernel(x):
    n, d = x.shape
    g = jnp.asarray(_anchor_indices(n))
    return _build_sc_call(n, d)(x, g)


# speedup vs baseline: 1.0137x; 1.0137x over previous
import functools

import jax
import jax.numpy as jnp
import numpy as np
from jax import lax
from jax.experimental import pallas as pl
from jax.experimental.pallas import tpu as pltpu
from jax.experimental.pallas import tpu_sc as plsc

_LANES = 16
_RING = 4
_CB = 8


@functools.lru_cache(maxsize=None)
def _build_sc_call(n, d):
    info = plsc.get_sparse_core_info()
    nc = info.num_cores
    nw = nc * info.num_subcores
    bpw = n // nw
    cb = _CB
    steps = bpw // cb
    assert steps % _RING == 0
    mesh = plsc.VectorSubcoreMesh(core_axis_name="c", subcore_axis_name="s")

    @functools.partial(
        pl.kernel,
        mesh=mesh,
        out_type=jax.ShapeDtypeStruct((n, 2 * d), jnp.float32),
        scratch_types=[
            pltpu.VMEM((bpw,), jnp.int32),
            pltpu.VMEM((_RING, cb, d), jnp.float32),
            pltpu.VMEM((_RING, cb, d), jnp.float32),
            pltpu.VMEM((_RING, cb, d), jnp.float32),
            pltpu.SemaphoreType.DMA((_RING,)),
            pltpu.SemaphoreType.DMA((_RING,)),
            pltpu.SemaphoreType.DMA((_RING,)),
            pltpu.SemaphoreType.DMA((_RING,)),
        ],
    )
    def call(x_hbm, g_hbm, out_hbm, idx_all, av, xv, dv, gsem, xsem, wasem, wdsem):
        wid = lax.axis_index("s") * nc + lax.axis_index("c")
        base = wid * bpw
        pltpu.sync_copy(g_hbm.at[pl.ds(base, bpw)], idx_all)

        def issue_loads(i, b):
            r0 = base + i * cb
            pltpu.async_copy(x_hbm.at[idx_all.at[pl.ds(i * cb, cb)]],
                             av.at[b], gsem.at[b])
            pltpu.async_copy(x_hbm.at[pl.ds(r0, cb)], xv.at[b], xsem.at[b])

        def wait_writes(b):
            pltpu.make_async_copy(
                av.at[b], out_hbm.at[pl.ds(0, cb), pl.ds(0, d)], wasem.at[b]
            ).wait()
            pltpu.make_async_copy(
                dv.at[b], out_hbm.at[pl.ds(0, cb), pl.ds(d, d)], wdsem.at[b]
            ).wait()

        for b in range(_RING - 1):
            issue_loads(b, b)

        def group(k, carry):
            for b in range(_RING):
                i = _RING * k + b
                bp = (b + _RING - 1) % _RING

                @pl.when(jnp.logical_and(i >= 1, i + (_RING - 1) < steps))
                def _():
                    wait_writes(bp)

                @pl.when(i + (_RING - 1) < steps)
                def _():
                    issue_loads(i + (_RING - 1), bp)

                pltpu.make_async_copy(
                    x_hbm.at[pl.ds(0, cb)], av.at[b], gsem.at[b]).wait()
                pltpu.make_async_copy(
                    x_hbm.at[pl.ds(0, cb)], xv.at[b], xsem.at[b]).wait()

                def row(r, rc):
                    for c in range(d // _LANES):
                        sl = pl.ds(c * _LANES, _LANES)
                        dv[b, r, sl] = xv[b, r, sl] - av[b, r, sl]
                    return rc

                lax.fori_loop(0, cb, row, 0)

                r0 = base + i * cb
                pltpu.async_copy(
                    av.at[b], out_hbm.at[pl.ds(r0, cb), pl.ds(0, d)], wasem.at[b])
                pltpu.async_copy(
                    dv.at[b], out_hbm.at[pl.ds(r0, cb), pl.ds(d, d)], wdsem.at[b])
            return carry

        lax.fori_loop(0, steps // _RING, group, 0)
        for b in range(_RING):
            wait_writes(b)

    return call


@functools.lru_cache(maxsize=None)
def _anchor_indices(n):
    with jax.ensure_compile_time_eval(), \
            jax.default_device(jax.devices("cpu")[0]):
        k1, k2 = jax.random.split(jax.random.key(42))
        perm = jax.random.permutation(k1, n)
        idx = jax.random.randint(k2, (n,), 0, n)
        g = jnp.take(perm, idx, axis=0)
        return np.asarray(g, dtype=np.int32)


def kernel(x):
    n, d = x.shape
    g = jnp.asarray(_anchor_indices(n))
    return _build_sc_call(n, d)(x, g)
